# baseline (speedup 1.0000x reference)
import functools

import jax
import jax.numpy as jnp
from jax import lax
from jax.experimental import pallas as pl
from jax.experimental.pallas import tpu as pltpu
from jax.experimental.pallas import tpu_sc as plsc

_NSPK = 1000
_EMBED = 128
_BATCH = 16384
_OFFSET = 120
_MOD = 240

_NC = 2
_NS = 16
_NW = _NC * _NS
_BPW = _BATCH // _NW
_CK = 128
_NCHUNK = (2 * _BPW) // _CK
_NBUF = 3
_BLK = 16

_mesh = plsc.VectorSubcoreMesh(core_axis_name="c", subcore_axis_name="s")


@functools.partial(
    pl.kernel,
    mesh=_mesh,
    out_type=(
        jax.ShapeDtypeStruct((_BATCH * _EMBED,), jnp.float32),
        jax.ShapeDtypeStruct((_BATCH * _EMBED,), jnp.float32),
    ),
    scratch_types=[
        pltpu.VMEM((_BPW,), jnp.int32),
        pltpu.VMEM((_MOD * _EMBED,), jnp.float32),
    ]
    + [pltpu.VMEM((_CK * _EMBED,), jnp.float32)] * _NBUF
    + [pltpu.SemaphoreType.DMA]
    + [pltpu.SemaphoreType.DMA] * _NBUF,
)
def _emb_lookup(idx_hbm, table_hbm, out_a, out_b, idx_v, table_v, *rest):
    bufs = rest[:_NBUF]
    semt = rest[_NBUF]
    semw = rest[_NBUF + 1 :]
    wid = lax.axis_index("s") * _NC + lax.axis_index("c")
    base = wid * _BPW

    stage = pltpu.async_copy(table_hbm.at[pl.ds(0, _MOD * _EMBED)], table_v, semt)
    pltpu.sync_copy(idx_hbm.at[wid], idx_v)
    stage.wait()

    def assemble(t):
        buf = bufs[t % _NBUF]

        def block(b, carry):
            rows16 = idx_v[pl.ds((t % 4) * _CK + b * _BLK, _BLK)]
            if t >= _NCHUNK // 2:
                rows16 = jnp.where(
                    rows16 >= _MOD - _OFFSET,
                    rows16 - (_MOD - _OFFSET),
                    rows16 + _OFFSET,
                )
            av = rows16 * _EMBED
            dst0 = b * (_BLK * _EMBED)
            for r in range(_BLK):
                src = av[r]
                for j in range(_EMBED // 16):
                    buf[pl.ds(dst0 + r * _EMBED + j * 16, 16)] = table_v[
                        pl.ds(src + j * 16, 16)
                    ]
            return carry

        lax.fori_loop(0, _CK // _BLK, block, 0)

    def write(t):
        s = t % _NBUF
        dst = out_a if t < _NCHUNK // 2 else out_b
        rows = pl.ds((base + (t % 4) * _CK) * _EMBED, _CK * _EMBED)
        return pltpu.async_copy(bufs[s], dst.at[rows], semw[s])

    wh = [None] * _NCHUNK
    for t in range(_NCHUNK):
        if t >= _NBUF:
            wh[t - _NBUF].wait()
        assemble(t)
        wh[t] = write(t)
    for t in range(_NCHUNK - _NBUF, _NCHUNK):
        wh[t].wait()


def kernel(spkr, table):
    idx = spkr.reshape(_NW, _BPW)
    ident, ident_n = _emb_lookup(idx, table.reshape(-1))
    return (
        ident.reshape(_BATCH, _EMBED),
        ident_n.reshape(_BATCH, _EMBED),
    )


# speedup vs baseline: 1.3528x; 1.3528x over previous
import functools

import jax
import jax.numpy as jnp
from jax import lax
from jax.experimental import pallas as pl
from jax.experimental.pallas import tpu as pltpu
from jax.experimental.pallas import tpu_sc as plsc

_NSPK = 1000
_EMBED = 128
_BATCH = 16384
_OFFSET = 120
_MOD = 240

_NC = 2
_NS = 16
_NW = _NC * _NS
_BPW = _BATCH // _NW
_CK = 64
_NCHUNK = _BPW // _CK
_NBUF = 7

_mesh = plsc.VectorSubcoreMesh(core_axis_name="c", subcore_axis_name="s")


@functools.partial(
    pl.kernel,
    mesh=_mesh,
    out_type=(
        jax.ShapeDtypeStruct((_BATCH, _EMBED), jnp.float32),
        jax.ShapeDtypeStruct((_BATCH, _EMBED), jnp.float32),
    ),
    scratch_types=[
        pltpu.VMEM((_BPW,), jnp.int32),
        pltpu.VMEM((_NBUF, _CK, 2 * _EMBED), jnp.float32),
    ]
    + [pltpu.SemaphoreType.DMA] * (2 * _NBUF),
)
def _emb_lookup(idx_hbm, comb_hbm, out_a, out_b, idx_v, bufs, *sems):
    semg = sems[:_NBUF]
    semw = sems[_NBUF:]
    wid = lax.axis_index("s") * _NC + lax.axis_index("c")
    base = wid * _BPW

    pltpu.sync_copy(idx_hbm.at[wid], idx_v)

    def gather(t):
        return pltpu.async_copy(
            comb_hbm.at[idx_v.at[pl.ds(t * _CK, _CK)]],
            bufs.at[t % _NBUF],
            semg[t % _NBUF],
        )

    def writes(t):
        s = t % _NBUF
        rows = pl.ds(base + t * _CK, _CK)
        wa = pltpu.async_copy(
            bufs.at[s, :, pl.ds(0, _EMBED)], out_a.at[rows], semw[s]
        )
        wb = pltpu.async_copy(
            bufs.at[s, :, pl.ds(_EMBED, _EMBED)], out_b.at[rows], semw[s]
        )
        return wa, wb

    gh = [None] * _NCHUNK
    wh = [None] * _NCHUNK
    for t in range(_NBUF):
        gh[t] = gather(t)
    for t in range(_NCHUNK):
        nxt = t + _NBUF - 1
        if _NBUF <= nxt < _NCHUNK:
            for h in wh[nxt - _NBUF]:
                h.wait()
            gh[nxt] = gather(nxt)
        gh[t].wait()
        wh[t] = writes(t)
    for t in range(max(0, _NCHUNK - _NBUF), _NCHUNK):
        for h in wh[t]:
            h.wait()


def kernel(spkr, table):
    idx = spkr.reshape(_NW, _BPW)
    tbl = table[:_MOD]
    comb = jnp.concatenate([tbl, jnp.roll(tbl, -_OFFSET, axis=0)], axis=1)
    ident, ident_n = _emb_lookup(idx, comb)
    return ident, ident_n


# speedup vs baseline: 4.0855x; 3.0201x over previous
import functools

import jax
import jax.numpy as jnp
from jax import lax
from jax.experimental import pallas as pl
from jax.experimental.pallas import tpu as pltpu

_NSPK = 1000
_EMBED = 128
_BATCH = 16384
_OFFSET = 120
_MOD = 240
_K = 256
_BBLK = 1024
_NBLK = _BATCH // _BBLK


def _tc_body(idx_ref, comb_ref, outa_ref, outb_ref):
    idx = idx_ref[0, 0, :]
    iota = lax.broadcasted_iota(jnp.int32, (_BBLK, _K), 1)
    onehot = (idx[:, None] == iota).astype(jnp.float32)
    out = jnp.dot(onehot, comb_ref[...], preferred_element_type=jnp.float32)
    outa_ref[...] = out[:, :_EMBED]
    outb_ref[...] = out[:, _EMBED:]


_tc_call = pl.pallas_call(
    _tc_body,
    grid=(_NBLK,),
    in_specs=[
        pl.BlockSpec((1, 1, _BBLK), lambda i: (i, 0, 0)),
        pl.BlockSpec((_K, 2 * _EMBED), lambda i: (0, 0)),
    ],
    out_specs=[
        pl.BlockSpec((_BBLK, _EMBED), lambda i: (i, 0)),
        pl.BlockSpec((_BBLK, _EMBED), lambda i: (i, 0)),
    ],
    out_shape=[
        jax.ShapeDtypeStruct((_BATCH, _EMBED), jnp.float32),
        jax.ShapeDtypeStruct((_BATCH, _EMBED), jnp.float32),
    ],
)


def kernel(spkr, table):
    idx = spkr.reshape(_NBLK, 1, _BBLK)
    tbl = table[:_MOD]
    comb = jnp.concatenate([tbl, jnp.roll(tbl, -_OFFSET, axis=0)], axis=1)
    comb = jnp.pad(comb, ((0, _K - _MOD), (0, 0)))
    ident, ident_n = _tc_call(idx, comb)
    return ident, ident_n
